# Initial kernel scaffold; baseline (speedup 1.0000x reference)
import jax
import jax.numpy as jnp
from jax.experimental import pallas as pl


def kernel(x, edge_index, edge_weight, batch, W1, b1, g1, be1, W2, b2, g2, be2, W3, b3, linW, linb):
    raise NotImplementedError("write your pallas kernel here")


# speedup vs baseline: 7.8376x; 7.8376x over previous
import functools

import jax
import jax.numpy as jnp
from jax import lax
from jax.experimental import pallas as pl
from jax.experimental.pallas import tpu as pltpu
from jax.experimental.pallas import tpu_sc as plsc

N_NODES = 10000
N_EDGES = 320000
DIM = 128
N_GRAPHS = 64
EPS = 1e-5

NC = 2
NS = 16
NW = NC * NS
CHUNK = 128
EW = -(-N_EDGES // (NW * CHUNK)) * CHUNK
NCHUNK = EW // CHUNK
E_PAD = NW * EW
NODES_PAD = 10240
ROWS_PER_TILE = NODES_PAD // NS

_mesh = plsc.VectorSubcoreMesh(core_axis_name="c", subcore_axis_name="s")


def _deg_body(dst_hbm, w_hbm, deg_out, dstb, wb, zb, deg_acc):
    cid = lax.axis_index("c")
    sid = lax.axis_index("s")
    wid = sid * NC + cid

    zero16 = jnp.zeros((16,), jnp.float32)

    def _z(i, _):
        zb[pl.ds(i * 16, 16)] = zero16
        return _

    lax.fori_loop(0, ROWS_PER_TILE // 16, _z, None)
    pltpu.sync_copy(zb, deg_acc.at[pl.ds(sid * ROWS_PER_TILE, ROWS_PER_TILE)])
    plsc.subcore_barrier()

    def _chunk(i, _):
        base = wid * EW + i * CHUNK
        pltpu.sync_copy(dst_hbm.at[pl.ds(base, CHUNK)], dstb)
        pltpu.sync_copy(w_hbm.at[pl.ds(base, CHUNK)], wb)
        pltpu.sync_copy(wb, deg_acc.at[dstb], add=True)
        return _

    lax.fori_loop(0, NCHUNK, _chunk, None)
    plsc.subcore_barrier()
    pltpu.sync_copy(
        deg_acc.at[pl.ds(sid * ROWS_PER_TILE, ROWS_PER_TILE)],
        deg_out.at[cid, pl.ds(sid * ROWS_PER_TILE, ROWS_PER_TILE)],
    )


_deg_kernel = pl.kernel(
    _deg_body,
    out_type=jax.ShapeDtypeStruct((NC, NODES_PAD), jnp.float32),
    mesh=_mesh,
    scratch_types=[
        pltpu.VMEM((CHUNK,), jnp.int32),
        pltpu.VMEM((CHUNK,), jnp.float32),
        pltpu.VMEM((ROWS_PER_TILE,), jnp.float32),
        pltpu.VMEM_SHARED((NODES_PAD,), jnp.float32),
    ],
)


def _spmv_body(u_hbm, src_hbm, dst_hbm, w_hbm, v_out, srcb, dstb, wb, rows, acc, gsem):
    cid = lax.axis_index("c")
    sid = lax.axis_index("s")
    wid = sid * NC + cid

    zero16 = jnp.zeros((16,), jnp.float32)

    def _zr(e, _):
        for c in range(DIM // 16):
            rows[e, pl.ds(c * 16, 16)] = zero16
        return _

    lax.fori_loop(0, CHUNK, _zr, None)
    r0 = sid * ROWS_PER_TILE
    for j in range(ROWS_PER_TILE // CHUNK):
        pltpu.sync_copy(rows, acc.at[pl.ds(r0 + j * CHUNK, CHUNK)])
    plsc.subcore_barrier()

    def _chunk(i, _):
        base = wid * EW + i * CHUNK
        pltpu.sync_copy(src_hbm.at[pl.ds(base, CHUNK)], srcb)
        pltpu.sync_copy(dst_hbm.at[pl.ds(base, CHUNK)], dstb)
        pltpu.sync_copy(w_hbm.at[pl.ds(base, CHUNK)], wb)
        pltpu.async_copy(u_hbm.at[srcb], rows, gsem).wait()

        def _mul(g, _):
            wv16 = wb[pl.ds(g * 16, 16)]
            for l in range(16):
                e = g * 16 + l
                wv = jnp.full((16,), wv16[l], jnp.float32)
                for c in range(DIM // 16):
                    rows[e, pl.ds(c * 16, 16)] = rows[e, pl.ds(c * 16, 16)] * wv
            return _

        lax.fori_loop(0, CHUNK // 16, _mul, None)
        pltpu.sync_copy(rows, acc.at[dstb], add=True)
        return _

    lax.fori_loop(0, NCHUNK, _chunk, None)
    plsc.subcore_barrier()
    pltpu.sync_copy(
        acc.at[pl.ds(r0, ROWS_PER_TILE)],
        v_out.at[cid, pl.ds(r0, ROWS_PER_TILE)],
    )


_spmv_kernel = pl.kernel(
    _spmv_body,
    out_type=jax.ShapeDtypeStruct((NC, NODES_PAD, DIM), jnp.float32),
    mesh=_mesh,
    scratch_types=[
        pltpu.VMEM((CHUNK,), jnp.int32),
        pltpu.VMEM((CHUNK,), jnp.int32),
        pltpu.VMEM((CHUNK,), jnp.float32),
        pltpu.VMEM((CHUNK, DIM), jnp.float32),
        pltpu.VMEM_SHARED((NODES_PAD, DIM), jnp.float32),
        pltpu.SemaphoreType.DMA,
    ],
)


def _stage0_body(x_ref, w1_ref, degp_ref, u0_ref, dinv_ref):
    deg = degp_ref[0, :N_NODES] + degp_ref[1, :N_NODES] + 1.0
    dinv = lax.rsqrt(deg)
    dinv_ref[...] = dinv[:, None]
    ids = lax.broadcasted_iota(jnp.int32, (N_NODES, DIM), 1)
    oh = (ids == x_ref[...][:, None]).astype(jnp.float32)
    h0 = jnp.dot(oh, w1_ref[...], preferred_element_type=jnp.float32)
    u0_ref[...] = dinv[:, None] * h0


_stage0 = pl.pallas_call(
    _stage0_body,
    out_shape=(
        jax.ShapeDtypeStruct((N_NODES, DIM), jnp.float32),
        jax.ShapeDtypeStruct((N_NODES, 1), jnp.float32),
    ),
)


def _stage_body(vp_ref, u_ref, dinv_ref, b_ref, g_ref, be_ref, w_ref, out_ref):
    v = vp_ref[0, :N_NODES, :] + vp_ref[1, :N_NODES, :]
    dinv = dinv_ref[...]
    t = dinv * (v + u_ref[...]) + b_ref[...][None, :]
    m = jnp.mean(t, axis=0)
    cdev = t - m[None, :]
    var = jnp.mean(cdev * cdev, axis=0)
    t = cdev * lax.rsqrt(var + EPS)[None, :] * g_ref[...][None, :] + be_ref[...][None, :]
    t = jnp.maximum(t, 0.0)
    h = jnp.dot(t, w_ref[...], preferred_element_type=jnp.float32)
    out_ref[...] = dinv * h


_stage = pl.pallas_call(
    _stage_body,
    out_shape=jax.ShapeDtypeStruct((N_NODES, DIM), jnp.float32),
)


def _final_body(vp_ref, u_ref, dinv_ref, b3_ref, batch_ref, linw_ref, linb_ref, out_ref):
    v = vp_ref[0, :N_NODES, :] + vp_ref[1, :N_NODES, :]
    h3 = dinv_ref[...] * (v + u_ref[...]) + b3_ref[...][None, :]
    gid = lax.broadcasted_iota(jnp.int32, (N_GRAPHS, N_NODES), 0)
    oht = (gid == batch_ref[...][None, :]).astype(jnp.float32)
    cnt = jnp.sum(oht, axis=1)
    summed = jnp.dot(oht, h3, preferred_element_type=jnp.float32)
    pooled = summed / jnp.maximum(cnt, 1.0)[:, None]
    out_ref[...] = (
        jnp.dot(pooled, linw_ref[...], preferred_element_type=jnp.float32)
        + linb_ref[...][None, :]
    )


_final = pl.pallas_call(
    _final_body,
    out_shape=jax.ShapeDtypeStruct((N_GRAPHS, DIM), jnp.float32),
)


def kernel(x, edge_index, edge_weight, batch, W1, b1, g1, be1, W2, b2, g2, be2,
           W3, b3, linW, linb):
    src = edge_index[0].astype(jnp.int32)
    dst = edge_index[1].astype(jnp.int32)
    w = edge_weight.astype(jnp.float32)
    pad = E_PAD - N_EDGES
    src_p = jnp.concatenate([src, jnp.zeros((pad,), jnp.int32)])
    dst_p = jnp.concatenate([dst, jnp.zeros((pad,), jnp.int32)])
    w_p = jnp.concatenate([w, jnp.zeros((pad,), jnp.float32)])

    degp = _deg_kernel(dst_p, w_p)
    u0, dinv = _stage0(x.astype(jnp.int32), W1, degp)
    vp1 = _spmv_kernel(u0, src_p, dst_p, w_p)
    u1 = _stage(vp1, u0, dinv, b1, g1, be1, W2)
    vp2 = _spmv_kernel(u1, src_p, dst_p, w_p)
    u2 = _stage(vp2, u1, dinv, b2, g2, be2, W3)
    vp3 = _spmv_kernel(u2, src_p, dst_p, w_p)
    return _final(vp3, u2, dinv, b3, batch.astype(jnp.int32), linW, linb)
